# baseline (speedup 1.0000x reference)
import jax
import jax.numpy as jnp
from jax.experimental import pallas as pl
from jax.experimental.pallas import tpu as pltpu

EPS = 1e-5
SLOPE = 0.01
TN_STATS = 8192
TN_ENC = 8192
TN_GATE = 8192


def _leaky(z):
    return jnp.where(z >= 0, z, SLOPE * z)


def kernel(x, conv_w, conv_b, gn1_w, gn1_b, codewords, scale, gn2_w, gn2_b, fc_w, fc_b, se_w, se_b):
    B, C, D, H, W = x.shape
    K = codewords.shape[0]
    nclass = se_w.shape[0]
    N = D * H * W
    GC = C // 4
    KG = K // 4
    cnt1 = float(GC * N)

    x3 = x.reshape(B, C, N)
    wb = conv_w.astype(jnp.bfloat16)
    cb_row = conv_b.reshape(1, C)
    g1w_row = gn1_w.reshape(1, C)
    g1b_row = gn1_b.reshape(1, C)
    scl_row = scale.reshape(1, K)
    fcb_row = fc_b.reshape(1, C)
    seb_row = se_b.reshape(1, nclass)
    g2w_full = jnp.broadcast_to(gn2_w[:, None], (K, C))
    g2b_full = jnp.broadcast_to(gn2_b[:, None], (K, C))

    params2 = pltpu.CompilerParams(
        dimension_semantics=("arbitrary", "arbitrary"))
    params1 = pltpu.CompilerParams(
        dimension_semantics=("arbitrary",))

    def _conv(xb_blk, w_ref, b_ref):
        h = jax.lax.dot_general(xb_blk, w_ref[...], (((0,), (1,)), ((), ())),
                                preferred_element_type=jnp.float32)
        return h + b_ref[...]

    def _stats_body(x_ref, w_ref, b_ref, xb_ref, s_ref):
        t = pl.program_id(1)
        xb_blk = x_ref[0].astype(jnp.bfloat16)
        xb_ref[0] = xb_blk
        h = _conv(xb_blk, w_ref, b_ref)
        ones = jnp.ones((1, TN_STATS), jnp.float32)
        cs1 = jax.lax.dot_general(ones, h, (((1,), (0,)), ((), ())),
                                  preferred_element_type=jnp.float32)
        cs2 = jax.lax.dot_general(ones, h * h, (((1,), (0,)), ((), ())),
                                  preferred_element_type=jnp.float32)
        upd = jnp.concatenate([cs1, cs2], axis=0)

        @pl.when(t == 0)
        def _():
            s_ref[0] = jnp.zeros((2, C), jnp.float32)

        s_ref[0] += upd

    xb3, stats = pl.pallas_call(
        _stats_body,
        grid=(B, N // TN_STATS),
        in_specs=[
            pl.BlockSpec((1, C, TN_STATS), lambda b, t: (b, 0, t)),
            pl.BlockSpec((C, C), lambda b, t: (0, 0)),
            pl.BlockSpec((1, C), lambda b, t: (0, 0)),
        ],
        out_specs=[
            pl.BlockSpec((1, C, TN_STATS), lambda b, t: (b, 0, t)),
            pl.BlockSpec((1, 2, C), lambda b, t: (b, 0, 0)),
        ],
        out_shape=[
            jax.ShapeDtypeStruct((B, C, N), jnp.bfloat16),
            jax.ShapeDtypeStruct((B, 2, C), jnp.float32),
        ],
        compiler_params=params2,
        name="enc_stats",
    )(x3, wb, cb_row)

    def _enc_body(xb_ref, w_ref, b_ref, g1w_ref, g1b_ref, s_ref, cw_ref,
                  scl_ref, E_ref):
        t = pl.program_id(1)
        h = _conv(xb_ref[0], w_ref, b_ref)
        s = s_ref[0]
        rvec_parts, mvec_parts = [], []
        for g in range(4):
            m = jnp.sum(s[0, g * GC:(g + 1) * GC]) / cnt1
            v = jnp.sum(s[1, g * GC:(g + 1) * GC]) / cnt1 - m * m
            r = jax.lax.rsqrt(v + EPS)
            rvec_parts.append(jnp.full((1, GC), r, jnp.float32))
            mvec_parts.append(jnp.full((1, GC), m, jnp.float32))
        rvec = jnp.concatenate(rvec_parts, axis=1)
        mvec = jnp.concatenate(mvec_parts, axis=1)
        alpha = g1w_ref[...] * rvec
        beta = g1b_ref[...] - mvec * alpha
        xf = _leaky(h * alpha + beta)

        cw = cw_ref[...]
        c2 = jax.lax.dot_general(jnp.ones((1, C), jnp.float32), cw * cw,
                                 (((1,), (1,)), ((), ())),
                                 preferred_element_type=jnp.float32)
        xc = jax.lax.dot_general(xf, cw, (((1,), (1,)), ((), ())),
                                 preferred_element_type=jnp.float32)
        x2 = jnp.sum(xf * xf, axis=1, keepdims=True)
        sl = scl_ref[...] * (x2 + (c2 - 2.0 * xc))
        mx = jnp.max(sl, axis=1, keepdims=True)
        e = jnp.exp(sl - mx)
        A = e / jnp.sum(e, axis=1, keepdims=True)

        Ep = jax.lax.dot_general(A, xf, (((0,), (0,)), ((), ())),
                                 preferred_element_type=jnp.float32)
        asum = jnp.sum(A, axis=0, keepdims=True)
        ii = jax.lax.broadcasted_iota(jnp.int32, (K, K), 0)
        jj = jax.lax.broadcasted_iota(jnp.int32, (K, K), 1)
        diagm = jnp.where(ii == jj, 1.0, 0.0) * asum
        corr = jnp.dot(diagm, cw, preferred_element_type=jnp.float32)

        @pl.when(t == 0)
        def _():
            E_ref[0] = jnp.zeros((K, C), jnp.float32)

        E_ref[0] += Ep - corr

    E_acc = pl.pallas_call(
        _enc_body,
        grid=(B, N // TN_ENC),
        in_specs=[
            pl.BlockSpec((1, C, TN_ENC), lambda b, t: (b, 0, t)),
            pl.BlockSpec((C, C), lambda b, t: (0, 0)),
            pl.BlockSpec((1, C), lambda b, t: (0, 0)),
            pl.BlockSpec((1, C), lambda b, t: (0, 0)),
            pl.BlockSpec((1, C), lambda b, t: (0, 0)),
            pl.BlockSpec((1, 2, C), lambda b, t: (b, 0, 0)),
            pl.BlockSpec((K, C), lambda b, t: (0, 0)),
            pl.BlockSpec((1, K), lambda b, t: (0, 0)),
        ],
        out_specs=pl.BlockSpec((1, K, C), lambda b, t: (b, 0, 0)),
        out_shape=jax.ShapeDtypeStruct((B, K, C), jnp.float32),
        compiler_params=params2,
        name="enc_encode",
    )(xb3, wb, cb_row, g1w_row, g1b_row, stats, codewords, scl_row)

    def _head_body(E_ref, g2w_ref, g2b_ref, fcw_ref, fcb_ref, sew_ref,
                   seb_ref, gam_ref, en_ref, se_ref):
        E = E_ref[0]
        blocks = []
        for g in range(4):
            blk = E[g * KG:(g + 1) * KG, :]
            m = jnp.mean(blk)
            v = jnp.mean(blk * blk) - m * m
            r = jax.lax.rsqrt(v + EPS)
            y = ((blk - m) * r) * g2w_ref[g * KG:(g + 1) * KG, :] \
                + g2b_ref[g * KG:(g + 1) * KG, :]
            blocks.append(_leaky(y))
        E2 = jnp.concatenate(blocks, axis=0)
        en = jnp.mean(E2, axis=0, keepdims=True)
        en_ref[0] = en
        gl = jax.lax.dot_general(en, fcw_ref[...], (((1,), (1,)), ((), ())),
                                 preferred_element_type=jnp.float32) + fcb_ref[...]
        gam_ref[0] = jax.nn.sigmoid(gl)
        sev = jax.lax.dot_general(en, sew_ref[...], (((1,), (1,)), ((), ())),
                                  preferred_element_type=jnp.float32) + seb_ref[...]
        se_ref[0] = sev

    gamma, en3, se3 = pl.pallas_call(
        _head_body,
        grid=(B,),
        in_specs=[
            pl.BlockSpec((1, K, C), lambda b: (b, 0, 0)),
            pl.BlockSpec((K, C), lambda b: (0, 0)),
            pl.BlockSpec((K, C), lambda b: (0, 0)),
            pl.BlockSpec((C, C), lambda b: (0, 0)),
            pl.BlockSpec((1, C), lambda b: (0, 0)),
            pl.BlockSpec((nclass, C), lambda b: (0, 0)),
            pl.BlockSpec((1, nclass), lambda b: (0, 0)),
        ],
        out_specs=[
            pl.BlockSpec((1, 1, C), lambda b: (b, 0, 0)),
            pl.BlockSpec((1, 1, C), lambda b: (b, 0, 0)),
            pl.BlockSpec((1, 1, nclass), lambda b: (b, 0, 0)),
        ],
        out_shape=[
            jax.ShapeDtypeStruct((B, 1, C), jnp.float32),
            jax.ShapeDtypeStruct((B, 1, C), jnp.float32),
            jax.ShapeDtypeStruct((B, 1, nclass), jnp.float32),
        ],
        compiler_params=params1,
        name="enc_head",
    )(E_acc, g2w_full, g2b_full, fc_w, fcb_row, se_w, seb_row)

    gamma_col = gamma.reshape(B, C, 1)

    def _gate_body(xb_ref, g_ref, out_ref):
        gcol = g_ref[0]
        o = xb_ref[0].astype(jnp.float32) * (1.0 + gcol)
        out_ref[0] = jnp.maximum(o, 0.0)

    out3 = pl.pallas_call(
        _gate_body,
        grid=(B, N // TN_GATE),
        in_specs=[
            pl.BlockSpec((1, C, TN_GATE), lambda b, t: (b, 0, t)),
            pl.BlockSpec((1, C, 1), lambda b, t: (b, 0, 0)),
        ],
        out_specs=pl.BlockSpec((1, C, TN_GATE), lambda b, t: (b, 0, t)),
        out_shape=jax.ShapeDtypeStruct((B, C, N), jnp.float32),
        compiler_params=params2,
        name="enc_gate",
    )(xb3, gamma_col)

    return (out3.reshape(B, C, D, H, W), en3.reshape(B, C), se3.reshape(B, nclass))


# speedup vs baseline: 1.0453x; 1.0453x over previous
import jax
import jax.numpy as jnp
from jax.experimental import pallas as pl
from jax.experimental.pallas import tpu as pltpu

EPS = 1e-5
SLOPE = 0.01
TN = 4096


def _leaky(z):
    return jnp.where(z >= 0, z, SLOPE * z)


def kernel(x, conv_w, conv_b, gn1_w, gn1_b, codewords, scale, gn2_w, gn2_b, fc_w, fc_b, se_w, se_b):
    B, C, D, H, W = x.shape
    K = codewords.shape[0]
    nclass = se_w.shape[0]
    N = D * H * W
    NT = N // TN
    GC = C // 4
    KG = K // 4
    cnt1 = float(GC * N)

    x3 = x.reshape(B, C, N)
    wb = conv_w.astype(jnp.bfloat16)
    cb_row = conv_b.reshape(1, C)
    g1w_row = gn1_w.reshape(1, C)
    g1b_row = gn1_b.reshape(1, C)
    scl_row = scale.reshape(1, K)
    fcb_row = fc_b.reshape(1, C)
    seb_row = se_b.reshape(1, nclass)
    g2w_full = jnp.broadcast_to(gn2_w[:, None], (K, C))
    g2b_full = jnp.broadcast_to(gn2_b[:, None], (K, C))

    def _body(x_ref, w_ref, b_ref, g1w_ref, g1b_ref, cw_ref, scl_ref,
              g2w_ref, g2b_ref, fcw_ref, fcb_ref, sew_ref, seb_ref,
              out_ref, en_ref, se_ref,
              xb_s, stat_s, E_s, gam_s):
        p = pl.program_id(1)
        t = pl.program_id(2)

        @pl.when(p == 0)
        def _phase0():
            xb_blk = x_ref[0].astype(jnp.bfloat16)
            xb_s[t] = xb_blk
            h = jax.lax.dot_general(xb_blk, w_ref[...], (((0,), (1,)), ((), ())),
                                    preferred_element_type=jnp.float32) + b_ref[...]
            ones = jnp.ones((1, TN), jnp.float32)
            cs1 = jax.lax.dot_general(ones, h, (((1,), (0,)), ((), ())),
                                      preferred_element_type=jnp.float32)
            cs2 = jax.lax.dot_general(ones, h * h, (((1,), (0,)), ((), ())),
                                      preferred_element_type=jnp.float32)
            upd = jnp.concatenate([cs1, cs2], axis=0)

            @pl.when(t == 0)
            def _():
                stat_s[...] = jnp.zeros((2, C), jnp.float32)

            stat_s[...] += upd

        @pl.when(p == 1)
        def _phase1():
            h = jax.lax.dot_general(xb_s[t], w_ref[...], (((0,), (1,)), ((), ())),
                                    preferred_element_type=jnp.float32) + b_ref[...]
            s = stat_s[...]
            rvec_parts, mvec_parts = [], []
            for g in range(4):
                m = jnp.sum(s[0, g * GC:(g + 1) * GC]) / cnt1
                v = jnp.sum(s[1, g * GC:(g + 1) * GC]) / cnt1 - m * m
                r = jax.lax.rsqrt(v + EPS)
                rvec_parts.append(jnp.full((1, GC), r, jnp.float32))
                mvec_parts.append(jnp.full((1, GC), m, jnp.float32))
            rvec = jnp.concatenate(rvec_parts, axis=1)
            mvec = jnp.concatenate(mvec_parts, axis=1)
            alpha = g1w_ref[...] * rvec
            beta = g1b_ref[...] - mvec * alpha
            xf = _leaky(h * alpha + beta)

            cw = cw_ref[...]
            c2 = jax.lax.dot_general(jnp.ones((1, C), jnp.float32), cw * cw,
                                     (((1,), (1,)), ((), ())),
                                     preferred_element_type=jnp.float32)
            xc = jax.lax.dot_general(xf, cw, (((1,), (1,)), ((), ())),
                                     preferred_element_type=jnp.float32)
            x2 = jnp.sum(xf * xf, axis=1, keepdims=True)
            sl = scl_ref[...] * (x2 + (c2 - 2.0 * xc))
            mx = jnp.max(sl, axis=1, keepdims=True)
            e = jnp.exp(sl - mx)
            A = e / jnp.sum(e, axis=1, keepdims=True)

            Ep = jax.lax.dot_general(A, xf, (((0,), (0,)), ((), ())),
                                     preferred_element_type=jnp.float32)
            asum = jnp.sum(A, axis=0, keepdims=True)
            ii = jax.lax.broadcasted_iota(jnp.int32, (K, K), 0)
            jj = jax.lax.broadcasted_iota(jnp.int32, (K, K), 1)
            diagm = jnp.where(ii == jj, 1.0, 0.0) * asum
            corr = jnp.dot(diagm, cw, preferred_element_type=jnp.float32)

            @pl.when(t == 0)
            def _():
                E_s[...] = jnp.zeros((K, C), jnp.float32)

            E_s[...] += Ep - corr

        @pl.when(p == 2)
        def _phase2():
            @pl.when(t == 0)
            def _head():
                E = E_s[...]
                blocks = []
                for g in range(4):
                    blk = E[g * KG:(g + 1) * KG, :]
                    m = jnp.mean(blk)
                    v = jnp.mean(blk * blk) - m * m
                    r = jax.lax.rsqrt(v + EPS)
                    y = ((blk - m) * r) * g2w_ref[g * KG:(g + 1) * KG, :] \
                        + g2b_ref[g * KG:(g + 1) * KG, :]
                    blocks.append(_leaky(y))
                E2 = jnp.concatenate(blocks, axis=0)
                en = jnp.mean(E2, axis=0, keepdims=True)
                en_ref[0] = en
                gl = jax.lax.dot_general(en, fcw_ref[...], (((1,), (1,)), ((), ())),
                                         preferred_element_type=jnp.float32) + fcb_ref[...]
                grow = 1.0 + jax.nn.sigmoid(gl)
                ic = jax.lax.broadcasted_iota(jnp.int32, (C, C), 0)
                jc = jax.lax.broadcasted_iota(jnp.int32, (C, C), 1)
                gam_s[...] = jnp.where(ic == jc, 1.0, 0.0) * grow
                sev = jax.lax.dot_general(en, sew_ref[...], (((1,), (1,)), ((), ())),
                                          preferred_element_type=jnp.float32) + seb_ref[...]
                se_ref[0] = sev

            o = jax.lax.dot_general(gam_s[...], xb_s[t].astype(jnp.float32),
                                    (((1,), (0,)), ((), ())),
                                    preferred_element_type=jnp.float32)
            out_ref[0] = jnp.maximum(o, 0.0)

    out3, en3, se3 = pl.pallas_call(
        _body,
        grid=(B, 3, NT),
        in_specs=[
            pl.BlockSpec((1, C, TN),
                         lambda b, p, t: (b, 0, jnp.where(p == 0, t, 0))),
            pl.BlockSpec((C, C), lambda b, p, t: (0, 0)),
            pl.BlockSpec((1, C), lambda b, p, t: (0, 0)),
            pl.BlockSpec((1, C), lambda b, p, t: (0, 0)),
            pl.BlockSpec((1, C), lambda b, p, t: (0, 0)),
            pl.BlockSpec((K, C), lambda b, p, t: (0, 0)),
            pl.BlockSpec((1, K), lambda b, p, t: (0, 0)),
            pl.BlockSpec((K, C), lambda b, p, t: (0, 0)),
            pl.BlockSpec((K, C), lambda b, p, t: (0, 0)),
            pl.BlockSpec((C, C), lambda b, p, t: (0, 0)),
            pl.BlockSpec((1, C), lambda b, p, t: (0, 0)),
            pl.BlockSpec((nclass, C), lambda b, p, t: (0, 0)),
            pl.BlockSpec((1, nclass), lambda b, p, t: (0, 0)),
        ],
        out_specs=[
            pl.BlockSpec((1, C, TN),
                         lambda b, p, t: (b, 0, jnp.where(p == 2, t, 0))),
            pl.BlockSpec((1, 1, C), lambda b, p, t: (b, 0, 0)),
            pl.BlockSpec((1, 1, nclass), lambda b, p, t: (b, 0, 0)),
        ],
        out_shape=[
            jax.ShapeDtypeStruct((B, C, N), jnp.float32),
            jax.ShapeDtypeStruct((B, 1, C), jnp.float32),
            jax.ShapeDtypeStruct((B, 1, nclass), jnp.float32),
        ],
        scratch_shapes=[
            pltpu.VMEM((NT, C, TN), jnp.bfloat16),
            pltpu.VMEM((2, C), jnp.float32),
            pltpu.VMEM((K, C), jnp.float32),
            pltpu.VMEM((C, C), jnp.float32),
        ],
        compiler_params=pltpu.CompilerParams(
            dimension_semantics=("arbitrary", "arbitrary", "arbitrary")),
        name="enc_fused",
    )(x3, wb, cb_row, g1w_row, g1b_row, codewords, scl_row,
      g2w_full, g2b_full, fc_w, fcb_row, se_w, seb_row)

    return (out3.reshape(B, C, D, H, W), en3.reshape(B, C), se3.reshape(B, nclass))


# speedup vs baseline: 1.0579x; 1.0120x over previous
import jax
import jax.numpy as jnp
from jax.experimental import pallas as pl
from jax.experimental.pallas import tpu as pltpu

EPS = 1e-5
SLOPE = 0.01
TN = 4096


def _leaky(z):
    return jnp.where(z >= 0, z, SLOPE * z)


def kernel(x, conv_w, conv_b, gn1_w, gn1_b, codewords, scale, gn2_w, gn2_b, fc_w, fc_b, se_w, se_b):
    B, C, D, H, W = x.shape
    K = codewords.shape[0]
    nclass = se_w.shape[0]
    N = D * H * W
    NT = N // TN
    GC = C // 4
    KG = K // 4
    cnt1 = float(GC * N)

    x3 = x.reshape(B, C, N)
    wb = conv_w.astype(jnp.bfloat16)
    cb_row = conv_b.reshape(1, C)
    g1w_row = gn1_w.reshape(1, C)
    g1b_row = gn1_b.reshape(1, C)
    scl_row = scale.reshape(1, K)
    fcb_row = fc_b.reshape(1, C)
    seb_row = se_b.reshape(1, nclass)
    g2w_full = jnp.broadcast_to(gn2_w[:, None], (K, C))
    g2b_full = jnp.broadcast_to(gn2_b[:, None], (K, C))

    def _body(x_ref, w_ref, b_ref, g1w_ref, g1b_ref, cw_ref, scl_ref,
              g2w_ref, g2b_ref, fcw_ref, fcb_ref, sew_ref, seb_ref,
              out_ref, en_ref, se_ref,
              xb_s, stat_s, E_s, gam_s):
        s = pl.program_id(0)
        t = pl.program_id(1)
        do_gate = (s == 2) | (s == 4)
        do_read = (s == 0) | (s == 2)
        do_enc = (s == 1) | (s == 3)

        @pl.when(do_gate)
        def _gate():
            @pl.when(t == 0)
            def _head():
                E = E_s[...]
                blocks = []
                for g in range(4):
                    blk = E[g * KG:(g + 1) * KG, :]
                    m = jnp.mean(blk)
                    v = jnp.mean(blk * blk) - m * m
                    r = jax.lax.rsqrt(v + EPS)
                    y = ((blk - m) * r) * g2w_ref[g * KG:(g + 1) * KG, :] \
                        + g2b_ref[g * KG:(g + 1) * KG, :]
                    blocks.append(_leaky(y))
                E2 = jnp.concatenate(blocks, axis=0)
                en = jnp.mean(E2, axis=0, keepdims=True)
                en_ref[0] = en
                gl = jax.lax.dot_general(en, fcw_ref[...], (((1,), (1,)), ((), ())),
                                         preferred_element_type=jnp.float32) + fcb_ref[...]
                grow = 1.0 + jax.nn.sigmoid(gl)
                ic = jax.lax.broadcasted_iota(jnp.int32, (C, C), 0)
                jc = jax.lax.broadcasted_iota(jnp.int32, (C, C), 1)
                gam_s[...] = jnp.where(ic == jc, 1.0, 0.0) * grow
                sev = jax.lax.dot_general(en, sew_ref[...], (((1,), (1,)), ((), ())),
                                          preferred_element_type=jnp.float32) + seb_ref[...]
                se_ref[0] = sev

            o = jax.lax.dot_general(gam_s[...], xb_s[t].astype(jnp.float32),
                                    (((1,), (0,)), ((), ())),
                                    preferred_element_type=jnp.float32)
            out_ref[0] = jnp.maximum(o, 0.0)

        @pl.when(do_read)
        def _read():
            xb_blk = x_ref[0].astype(jnp.bfloat16)
            xb_s[t] = xb_blk
            h = jax.lax.dot_general(xb_blk, w_ref[...], (((0,), (1,)), ((), ())),
                                    preferred_element_type=jnp.float32) + b_ref[...]
            ones = jnp.ones((1, TN), jnp.float32)
            cs1 = jax.lax.dot_general(ones, h, (((1,), (0,)), ((), ())),
                                      preferred_element_type=jnp.float32)
            cs2 = jax.lax.dot_general(ones, h * h, (((1,), (0,)), ((), ())),
                                      preferred_element_type=jnp.float32)
            upd = jnp.concatenate([cs1, cs2], axis=0)

            @pl.when(t == 0)
            def _():
                stat_s[...] = jnp.zeros((2, C), jnp.float32)

            stat_s[...] += upd

        @pl.when(do_enc)
        def _enc():
            h = jax.lax.dot_general(xb_s[t], w_ref[...], (((0,), (1,)), ((), ())),
                                    preferred_element_type=jnp.float32) + b_ref[...]
            sm = stat_s[...]
            rvec_parts, mvec_parts = [], []
            for g in range(4):
                m = jnp.sum(sm[0, g * GC:(g + 1) * GC]) / cnt1
                v = jnp.sum(sm[1, g * GC:(g + 1) * GC]) / cnt1 - m * m
                r = jax.lax.rsqrt(v + EPS)
                rvec_parts.append(jnp.full((1, GC), r, jnp.float32))
                mvec_parts.append(jnp.full((1, GC), m, jnp.float32))
            rvec = jnp.concatenate(rvec_parts, axis=1)
            mvec = jnp.concatenate(mvec_parts, axis=1)
            alpha = g1w_ref[...] * rvec
            beta = g1b_ref[...] - mvec * alpha
            xf = _leaky(h * alpha + beta)

            cw = cw_ref[...]
            c2 = jax.lax.dot_general(jnp.ones((1, C), jnp.float32), cw * cw,
                                     (((1,), (1,)), ((), ())),
                                     preferred_element_type=jnp.float32)
            xc = jax.lax.dot_general(xf, cw, (((1,), (1,)), ((), ())),
                                     preferred_element_type=jnp.float32)
            x2 = jnp.sum(xf * xf, axis=1, keepdims=True)
            sl = scl_ref[...] * (x2 + (c2 - 2.0 * xc))
            mx = jnp.max(sl, axis=1, keepdims=True)
            e = jnp.exp(sl - mx)
            A = e / jnp.sum(e, axis=1, keepdims=True)

            Ep = jax.lax.dot_general(A, xf, (((0,), (0,)), ((), ())),
                                     preferred_element_type=jnp.float32)
            asum = jnp.sum(A, axis=0, keepdims=True)
            ii = jax.lax.broadcasted_iota(jnp.int32, (K, K), 0)
            jj = jax.lax.broadcasted_iota(jnp.int32, (K, K), 1)
            diagm = jnp.where(ii == jj, 1.0, 0.0) * asum
            corr = jnp.dot(diagm, cw, preferred_element_type=jnp.float32)

            @pl.when(t == 0)
            def _():
                E_s[...] = jnp.zeros((K, C), jnp.float32)

            E_s[...] += Ep - corr

    def _x_map(s, t):
        return (jnp.where(s == 2, 1, 0), 0,
                jnp.where((s == 0) | (s == 2), t, 0))

    def _out_map(s, t):
        return (jnp.where(s >= 3, 1, 0), 0,
                jnp.where(s <= 1, 0,
                          jnp.where(s == 3, NT - 1, t)))

    def _head_map(s, t):
        return (jnp.where(s >= 3, 1, 0), 0, 0)

    const = lambda s, t: (0, 0)

    out3, en3, se3 = pl.pallas_call(
        _body,
        grid=(5, NT),
        in_specs=[
            pl.BlockSpec((1, C, TN), _x_map),
            pl.BlockSpec((C, C), const),
            pl.BlockSpec((1, C), const),
            pl.BlockSpec((1, C), const),
            pl.BlockSpec((1, C), const),
            pl.BlockSpec((K, C), const),
            pl.BlockSpec((1, K), const),
            pl.BlockSpec((K, C), const),
            pl.BlockSpec((K, C), const),
            pl.BlockSpec((C, C), const),
            pl.BlockSpec((1, C), const),
            pl.BlockSpec((nclass, C), const),
            pl.BlockSpec((1, nclass), const),
        ],
        out_specs=[
            pl.BlockSpec((1, C, TN), _out_map),
            pl.BlockSpec((1, 1, C), _head_map),
            pl.BlockSpec((1, 1, nclass), _head_map),
        ],
        out_shape=[
            jax.ShapeDtypeStruct((B, C, N), jnp.float32),
            jax.ShapeDtypeStruct((B, 1, C), jnp.float32),
            jax.ShapeDtypeStruct((B, 1, nclass), jnp.float32),
        ],
        scratch_shapes=[
            pltpu.VMEM((NT, C, TN), jnp.bfloat16),
            pltpu.VMEM((2, C), jnp.float32),
            pltpu.VMEM((K, C), jnp.float32),
            pltpu.VMEM((C, C), jnp.float32),
        ],
        compiler_params=pltpu.CompilerParams(
            dimension_semantics=("arbitrary", "arbitrary")),
        name="enc_fused",
    )(x3, wb, cb_row, g1w_row, g1b_row, codewords, scl_row,
      g2w_full, g2b_full, fc_w, fcb_row, se_w, seb_row)

    return (out3.reshape(B, C, D, H, W), en3.reshape(B, C), se3.reshape(B, nclass))
